# baseline (speedup 1.0000x reference)
import jax
import jax.numpy as jnp
from jax.experimental import pallas as pl

_W = 128
_BR = 2048


def _nm24_body(x_ref, o_ref):
    x = x_ref[...]
    lane = jax.lax.broadcasted_iota(jnp.uint32, x.shape, 1)
    p = lane & 3

    bits = jax.lax.bitcast_convert_type(x, jnp.uint32)
    key = ((bits & jnp.uint32(0x7FFFFFFF)) << 1) | (p < 2).astype(jnp.uint32)

    b = []
    for e in (1, 2, 3):
        idx = ((lane & ~jnp.uint32(3)) | ((lane + e) & 3)).astype(jnp.int32)
        mate = jnp.take_along_axis(key, idx, axis=1)
        b.append(mate > key if e < 3 else mate >= key)
    b1, b2, b3 = b
    drop = (b1 & b2) | (b1 & b3) | (b2 & b3)
    o_ref[...] = jnp.where(drop, jnp.zeros_like(x), x)


def _nm24(weights):
    m, n = weights.shape
    flat = weights.reshape(m * n // _W, _W)
    grid = (flat.shape[0] // _BR,)
    out = pl.pallas_call(
        _nm24_body,
        grid=grid,
        in_specs=[pl.BlockSpec((_BR, _W), lambda i: (i, 0))],
        out_specs=pl.BlockSpec((_BR, _W), lambda i: (i, 0)),
        out_shape=jax.ShapeDtypeStruct(flat.shape, weights.dtype),
    )(flat)
    return out.reshape(m, n)


@jax.jit
def kernel(weights):
    return _nm24(weights)


# speedup vs baseline: 3.4171x; 3.4171x over previous
import jax
import jax.numpy as jnp
from jax.experimental import pallas as pl

_BM = 256


def _nm24_body(x_ref, o_ref):
    n = x_ref.shape[1]
    shape = (x_ref.shape[0], 128)
    lane = jax.lax.broadcasted_iota(jnp.uint32, shape, 1)
    p = lane & 3
    tie = (p < 2).astype(jnp.uint32)
    perms = [((lane & ~jnp.uint32(3)) | ((lane + e) & 3)).astype(jnp.int32)
             for e in (1, 2, 3)]

    for c in range(n // 128):
        x = x_ref[:, c * 128:(c + 1) * 128]
        bits = jax.lax.bitcast_convert_type(x, jnp.uint32)
        key = ((bits & jnp.uint32(0x7FFFFFFF)) << 1) | tie
        mates = [jnp.take_along_axis(key, idx, axis=1) for idx in perms]
        b1 = mates[0] > key
        b2 = mates[1] > key
        b3 = mates[2] >= key
        drop = (b1 & b2) | (b1 & b3) | (b2 & b3)
        o_ref[:, c * 128:(c + 1) * 128] = jnp.where(drop, jnp.zeros_like(x), x)


def _nm24(weights):
    m, n = weights.shape
    grid = (m // _BM,)
    return pl.pallas_call(
        _nm24_body,
        grid=grid,
        in_specs=[pl.BlockSpec((_BM, n), lambda i: (i, 0))],
        out_specs=pl.BlockSpec((_BM, n), lambda i: (i, 0)),
        out_shape=jax.ShapeDtypeStruct((m, n), weights.dtype),
    )(weights)


@jax.jit
def kernel(weights):
    return _nm24(weights)


# speedup vs baseline: 3.5438x; 1.0371x over previous
import jax
import jax.numpy as jnp
from jax.experimental import pallas as pl

_BM = 512


def _nm24_body(x_ref, o_ref):
    n = x_ref.shape[1]
    shape = (x_ref.shape[0], 128)
    lane = jax.lax.broadcasted_iota(jnp.uint32, shape, 1)
    p = lane & 3
    tie = (p < 2).astype(jnp.uint32)
    perms = [((lane & ~jnp.uint32(3)) | ((lane + e) & 3)).astype(jnp.int32)
             for e in (1, 2, 3)]

    for c in range(n // 128):
        x = x_ref[:, c * 128:(c + 1) * 128]
        bits = jax.lax.bitcast_convert_type(x, jnp.uint32)
        key = (bits << 1) | tie
        m1 = jnp.take_along_axis(key, perms[0], axis=1)
        m2 = jnp.take_along_axis(key, perms[1], axis=1)
        m3 = jnp.take_along_axis(key, perms[2], axis=1)
        b1 = m1 > key
        b2 = m2 > key
        b3 = m3 >= key
        drop = (b1 & b2) | ((b1 | b2) & b3)
        o_ref[:, c * 128:(c + 1) * 128] = jnp.where(drop, jnp.zeros_like(x), x)


def _nm24(weights):
    m, n = weights.shape
    grid = (m // _BM,)
    return pl.pallas_call(
        _nm24_body,
        grid=grid,
        in_specs=[pl.BlockSpec((_BM, n), lambda i: (i, 0))],
        out_specs=pl.BlockSpec((_BM, n), lambda i: (i, 0)),
        out_shape=jax.ShapeDtypeStruct((m, n), weights.dtype),
    )(weights)


@jax.jit
def kernel(weights):
    return _nm24(weights)
